# baseline (speedup 1.0000x reference)
import jax
import jax.numpy as jnp
from jax.experimental import pallas as pl
from jax.experimental.pallas import tpu as pltpu

_H = 512
_W = 512
_STRIP = 16
_IPB = 16


def _erode_pair(xs, prev_row):
    xu = jnp.concatenate([prev_row, xs[:-1]], axis=0)
    r = jnp.minimum(xs, xu)
    rm = jnp.concatenate([r[:, :1], r[:, :-1]], axis=1)
    rp = jnp.concatenate([r[:, 1:], r[:, _W - 2:_W - 1]], axis=1)
    return jnp.minimum(r, rm), jnp.minimum(r, rp)


def _dilate_sqerr(g, g_row, x):
    gd = jnp.concatenate([g[1:], g_row], axis=0)
    opened = jnp.maximum(g, gd)
    diff = x - opened
    return diff * diff


def _fold(acc, d2):
    for m in range(d2.shape[0] // 8):
        acc = acc + d2[8 * m:8 * m + 8]
    return acc


def _opening_mse_body(x_ref, out_ref):
    j = pl.program_id(1)
    n_strips = _H // _STRIP

    def img_body(k, acc):
        x_prev = g_prev = None
        for s in range(n_strips):
            xs = x_ref[k, s * _STRIP:(s + 1) * _STRIP, :]
            if s == 0:
                prev_row = xs[0:1]
            else:
                prev_row = x_prev[_STRIP - 1:_STRIP]
            e, en = _erode_pair(xs, prev_row)
            g = jnp.maximum(e, en)
            if s > 0:
                acc = _fold(acc, _dilate_sqerr(g_prev, g[0:1], x_prev))
            x_prev, g_prev = xs, g
        last = _STRIP - 1
        return _fold(acc, _dilate_sqerr(
            g_prev, g_prev[last:last + 1], x_prev))

    acc = jax.lax.fori_loop(
        0, _IPB, img_body, jnp.zeros((8, _W), jnp.float32))
    total = jnp.sum(acc).reshape(1, 1, 1)

    @pl.when(j == 0)
    def _():
        out_ref[...] = total

    @pl.when(j != 0)
    def _():
        out_ref[...] = out_ref[...] + total


def kernel(labels):
    b, c, h, w = labels.shape
    n = b * c
    x = labels.reshape(n, h, w)
    per_core = n // 2 // _IPB
    partials = pl.pallas_call(
        _opening_mse_body,
        grid=(2, per_core),
        in_specs=[pl.BlockSpec((_IPB, h, w),
                               lambda i, j: (i * per_core + j, 0, 0))],
        out_specs=pl.BlockSpec((1, 1, 1), lambda i, j: (i, 0, 0)),
        out_shape=jax.ShapeDtypeStruct((2, 1, 1), jnp.float32),
        compiler_params=pltpu.CompilerParams(
            dimension_semantics=("parallel", "arbitrary"),
        ),
    )(x)
    return jnp.sum(partials) / (n * h * w)


# speedup vs baseline: 1.0122x; 1.0122x over previous
import jax
import jax.numpy as jnp
from jax.experimental import pallas as pl
from jax.experimental.pallas import tpu as pltpu

_H = 512
_W = 512
_STRIP = 16
_IPB = 8


def _erode_pair(xs, prev_row):
    xu = jnp.concatenate([prev_row, xs[:-1]], axis=0)
    r = jnp.minimum(xs, xu)
    rm = jnp.concatenate([r[:, :1], r[:, :-1]], axis=1)
    rp = jnp.concatenate([r[:, 1:], r[:, _W - 2:_W - 1]], axis=1)
    return jnp.minimum(r, rm), jnp.minimum(r, rp)


def _dilate_sqerr(g, g_row, x):
    gd = jnp.concatenate([g[1:], g_row], axis=0)
    opened = jnp.maximum(g, gd)
    diff = x - opened
    return diff * diff


def _fold(acc, d2):
    for m in range(d2.shape[0] // 8):
        acc = acc + d2[8 * m:8 * m + 8]
    return acc


def _opening_mse_body(x_ref, out_ref):
    j = pl.program_id(1)
    n_strips = _H // _STRIP

    def img_body(k, acc):
        x_prev = g_prev = None
        for s in range(n_strips):
            xs = x_ref[k, s * _STRIP:(s + 1) * _STRIP, :]
            if s == 0:
                prev_row = xs[0:1]
            else:
                prev_row = x_prev[_STRIP - 1:_STRIP]
            e, en = _erode_pair(xs, prev_row)
            g = jnp.maximum(e, en)
            if s > 0:
                acc = _fold(acc, _dilate_sqerr(g_prev, g[0:1], x_prev))
            x_prev, g_prev = xs, g
        last = _STRIP - 1
        return _fold(acc, _dilate_sqerr(
            g_prev, g_prev[last:last + 1], x_prev))

    acc = jax.lax.fori_loop(
        0, _IPB, img_body, jnp.zeros((8, _W), jnp.float32))
    total = jnp.sum(acc).reshape(1, 1, 1)

    @pl.when(j == 0)
    def _():
        out_ref[...] = total

    @pl.when(j != 0)
    def _():
        out_ref[...] = out_ref[...] + total


def kernel(labels):
    b, c, h, w = labels.shape
    n = b * c
    x = labels.reshape(n, h, w)
    per_core = n // 2 // _IPB
    partials = pl.pallas_call(
        _opening_mse_body,
        grid=(2, per_core),
        in_specs=[pl.BlockSpec((_IPB, h, w),
                               lambda i, j: (i * per_core + j, 0, 0))],
        out_specs=pl.BlockSpec((1, 1, 1), lambda i, j: (i, 0, 0)),
        out_shape=jax.ShapeDtypeStruct((2, 1, 1), jnp.float32),
        compiler_params=pltpu.CompilerParams(
            dimension_semantics=("parallel", "arbitrary"),
            flags={"XLA_TPU_STORE_TO_LOAD_FORWARDING_WINDOW": 12288},
        ),
    )(x)
    return jnp.sum(partials) / (n * h * w)
